# baseline (speedup 1.0000x reference)
import functools

import jax
import jax.numpy as jnp
import numpy as np
from jax import lax
from jax.experimental import pallas as pl
from jax.experimental.pallas import tpu as pltpu
from jax.experimental.pallas import tpu_sc as plsc

B = 4096
L = 128
F = 144
K = 64

NCHUNK = 4
CROWS = B // NCHUNK


_NW = 32
_RPW = CROWS // _NW
_G = 16
_NG = _RPW // _G


def _scan_feats(load_feat, nfeat, k0, len_r):
    ilane = lax.iota(jnp.int32, 16)
    nchunks = (len_r + 15) // 16

    def chunk_body(tc, accs):
        t0 = tc * 16
        m = (t0 + ilane) < len_r
        return tuple(
            jnp.maximum(accs[f], jnp.where(m, load_feat(f, t0), 0.0))
            for f in range(nfeat))

    accs = lax.fori_loop(0, nchunks, chunk_body,
                         tuple(jnp.zeros((16,), jnp.float32)
                               for _ in range(nfeat)))
    fmax = jnp.zeros((16,), jnp.float32)
    for f in range(nfeat):
        fmax = jnp.where(accs[f] > 0.5,
                         jnp.maximum(fmax, float(k0 + f + 1)), fmax)
    return jnp.max(fmax, axis=0)


def _sc_base_body(cid, seqt_hbm, lens_hbm, out_hbm,
                  buf0, buf1, fbuf, lens_v, out_v, sem0, sem1, fsem):
    info = plsc.get_sparse_core_info()
    nc = info.num_cores
    wid = lax.axis_index("s") * nc + lax.axis_index("c")
    wbase = cid * CROWS + wid * _RPW

    pltpu.sync_copy(lens_hbm.at[pl.ds(wbase, _RPW)], lens_v)

    def start(g, buf, sem):
        r0 = wbase + g * _G
        return pltpu.async_copy(
            seqt_hbm.at[pl.ds(r0, _G), pl.ds(K - 16, 16), :], buf, sem)

    def process(g, buf):
        lens16 = lens_v[pl.ds(g * _G, _G)].astype(jnp.float32)
        ilane = lax.iota(jnp.int32, 16)

        def row_body(r16, bvec):
            len_r = jnp.max(jnp.where(ilane == r16, lens16, 0.0),
                            axis=0).astype(jnp.int32)
            lm3 = _scan_feats(
                lambda f, t0: buf[r16, f, pl.ds(t0, 16)], 16, K - 16, len_r)

            def fallback():
                row = wbase + g * _G + r16
                pltpu.async_copy(
                    seqt_hbm.at[row, pl.ds(0, K - 16), :], fbuf, fsem).wait()
                return _scan_feats(
                    lambda f, t0: fbuf[f, pl.ds(t0, 16)], K - 16, 0, len_r)

            base_r = lax.cond(lm3 > 0.0, lambda: lm3, fallback)
            return jnp.where(ilane == r16, base_r, bvec)

        bvec = lax.fori_loop(0, _G, row_body, jnp.zeros((16,), jnp.float32))
        out_v[pl.ds(g * _G, _G)] = bvec

    start(0, buf0, sem0).wait()
    c1 = start(1, buf1, sem1)
    process(0, buf0)
    c1.wait()
    process(1, buf1)

    pltpu.sync_copy(out_v, out_hbm.at[pl.ds(wid * _RPW, _RPW)])


def _sc_base_chunk(seqt, lens, cid):
    mesh = plsc.VectorSubcoreMesh(core_axis_name="c", subcore_axis_name="s")
    return pl.kernel(
        functools.partial(_sc_base_body, cid),
        out_type=jax.ShapeDtypeStruct((CROWS,), jnp.float32),
        mesh=mesh,
        scratch_types=[
            pltpu.VMEM((_G, 16, L), jnp.float32),
            pltpu.VMEM((_G, 16, L), jnp.float32),
            pltpu.VMEM((K - 16, L), jnp.float32),
            pltpu.VMEM((_RPW,), jnp.int32),
            pltpu.VMEM((_RPW,), jnp.float32),
            pltpu.SemaphoreType.DMA,
            pltpu.SemaphoreType.DMA,
            pltpu.SemaphoreType.DMA,
        ],
        compiler_params=pltpu.CompilerParams(needs_layout_passes=False),
    )(seqt, lens)


BB = 256

_TRI = np.triu(np.ones((L, L), np.float32)).astype(jnp.bfloat16)


def _writer_body(prev_ref, base_ref, logit0_ref, tri_ref, out_ref):
    del prev_ref
    is_new = logit0_ref[:, 0, :] > 0.0
    order = jax.lax.dot_general(
        is_new.astype(jnp.bfloat16), tri_ref[...], (((1,), (0,)), ((), ())),
        preferred_element_type=jnp.float32)
    nid = base_ref[:, :] + order - 1.0
    nid = jnp.where(is_new, nid, -1.0)
    kio = jax.lax.broadcasted_iota(jnp.int32, (BB, K, L), 1).astype(
        jnp.float32)
    out_ref[...] = (nid[:, None, :] == kio).astype(jnp.float32)


def _writer_chunk(prev_out, base2d_c, logitst, cid):
    grid = CROWS // BB
    step0 = cid * grid
    return pl.pallas_call(
        _writer_body,
        grid=(grid,),
        in_specs=[
            pl.BlockSpec(memory_space=pl.ANY),
            pl.BlockSpec((BB, 1), lambda i: (i, 0)),
            pl.BlockSpec((BB, 2, L), lambda i, s0=step0: (i + s0, 0, 0)),
            pl.BlockSpec((L, L), lambda i: (0, 0)),
        ],
        out_specs=pl.BlockSpec((BB, K, L), lambda i, s0=step0: (i + s0, 0, 0)),
        out_shape=jax.ShapeDtypeStruct((B, K, L), jnp.float32),
        input_output_aliases={0: 0},
    )(prev_out, base2d_c, logitst, jnp.asarray(_TRI))


def kernel(seq, enref_seq_len, is_new_logits):
    lens = enref_seq_len.astype(jnp.int32)
    seqt = jnp.swapaxes(seq, 1, 2)
    logitst = jnp.swapaxes(is_new_logits, 1, 2)

    bases = [_sc_base_chunk(seqt, lens, c) for c in range(NCHUNK)]
    out_t = jnp.empty((B, K, L), jnp.float32)
    for c in range(NCHUNK):
        out_t = _writer_chunk(out_t, bases[c].reshape(CROWS, 1), logitst, c)

    out = jnp.swapaxes(out_t, 1, 2)
    return jax.lax.stop_gradient(out)


# speedup vs baseline: 1.8758x; 1.8758x over previous
import jax
import jax.numpy as jnp
import numpy as np
from jax import lax
from jax.experimental import pallas as pl
from jax.experimental.pallas import tpu as pltpu
from jax.experimental.pallas import tpu_sc as plsc

B = 4096
L = 128
F = 144
K = 64


_NW = 32
_RPW = B // _NW
_G = 16
_TOP = 8
_NG = _RPW // _G


def _scan_feats(load_feat, nfeat, k0, len_r):
    ilane = lax.iota(jnp.int32, 16)
    nfull = len_r // 16

    def chunk_body(tc, accs):
        t0 = tc * 16
        return tuple(
            jnp.maximum(accs[f], load_feat(f, t0))
            for f in range(nfeat))

    accs = lax.fori_loop(0, nfull, chunk_body,
                         tuple(jnp.zeros((16,), jnp.float32)
                               for _ in range(nfeat)))
    t0 = nfull * 16
    m = (t0 + ilane) < len_r
    accs = tuple(
        jnp.maximum(accs[f], jnp.where(m, load_feat(f, jnp.minimum(t0, L - 16)), 0.0))
        for f in range(nfeat))
    fmax = jnp.zeros((16,), jnp.float32)
    for f in range(nfeat):
        fmax = jnp.where(accs[f] > 0.5,
                         jnp.maximum(fmax, float(k0 + f + 1)), fmax)
    return jnp.max(fmax, axis=0)


def _sc_base_body(seqt_hbm, lens_hbm, out_hbm,
                  buf0, buf1, fbuf, lens_v, out_v, sem0, sem1, fsem):
    info = plsc.get_sparse_core_info()
    nc = info.num_cores
    wid = lax.axis_index("s") * nc + lax.axis_index("c")
    wbase = wid * _RPW

    pltpu.sync_copy(lens_hbm.at[pl.ds(wbase, _RPW)], lens_v)

    def start(g, buf, sem):
        r0 = wbase + g * _G
        return pltpu.async_copy(
            seqt_hbm.at[pl.ds(r0, _G), pl.ds(K - _TOP, _TOP), :], buf, sem)

    def process(g, buf):
        lens16 = lens_v[pl.ds(g * _G, _G)].astype(jnp.float32)
        ilane = lax.iota(jnp.int32, 16)

        def row_body(r16, bvec):
            len_r = jnp.max(jnp.where(ilane == r16, lens16, 0.0),
                            axis=0).astype(jnp.int32)
            lm3 = _scan_feats(
                lambda f, t0: buf[r16, f, pl.ds(t0, 16)], _TOP, K - _TOP, len_r)

            def fallback():
                row = wbase + g * _G + r16
                pltpu.async_copy(
                    seqt_hbm.at[row, pl.ds(0, K - _TOP), :], fbuf, fsem).wait()
                return _scan_feats(
                    lambda f, t0: fbuf[f, pl.ds(t0, 16)], K - _TOP, 0, len_r)

            base_r = lax.cond(lm3 > 0.0, lambda: lm3, fallback)
            return jnp.where(ilane == r16, base_r, bvec)

        bvec = lax.fori_loop(0, _G, row_body, jnp.zeros((16,), jnp.float32))
        out_v[pl.ds(g * _G, _G)] = bvec

    start(0, buf0, sem0).wait()
    for p in range(_NG // 2):
        g0 = 2 * p
        c1 = start(g0 + 1, buf1, sem1)
        process(g0, buf0)
        c1.wait()
        c0 = start(min(g0 + 2, _NG - 1), buf0, sem0)
        process(g0 + 1, buf1)
        c0.wait()

    pltpu.sync_copy(out_v, out_hbm.at[pl.ds(wbase, _RPW)])


@jax.jit
def _sc_base(seqt, lens):
    mesh = plsc.VectorSubcoreMesh(core_axis_name="c", subcore_axis_name="s")
    return pl.kernel(
        _sc_base_body,
        out_type=jax.ShapeDtypeStruct((B,), jnp.float32),
        mesh=mesh,
        scratch_types=[
            pltpu.VMEM((_G, _TOP, L), jnp.float32),
            pltpu.VMEM((_G, _TOP, L), jnp.float32),
            pltpu.VMEM((K - _TOP, L), jnp.float32),
            pltpu.VMEM((_RPW,), jnp.int32),
            pltpu.VMEM((_RPW,), jnp.float32),
            pltpu.SemaphoreType.DMA,
            pltpu.SemaphoreType.DMA,
            pltpu.SemaphoreType.DMA,
        ],
        compiler_params=pltpu.CompilerParams(needs_layout_passes=False),
    )(seqt, lens)


BB = 256

_TRI = np.triu(np.ones((L, L), np.float32)).astype(jnp.bfloat16)


def _writer_body(base_ref, logit0_ref, tri_ref, out_ref):
    is_new = logit0_ref[:, 0, :] > 0.0
    order = jax.lax.dot_general(
        is_new.astype(jnp.bfloat16), tri_ref[...], (((1,), (0,)), ((), ())),
        preferred_element_type=jnp.float32)
    nid = base_ref[:, :] + order - 1.0
    nid = jnp.where(is_new, nid, -1.0)
    kio = jax.lax.broadcasted_iota(jnp.int32, (BB, K, L), 1).astype(
        jnp.float32)
    out_ref[...] = (nid[:, None, :] == kio).astype(jnp.float32)


def _writer(base2d, logitst):
    grid = B // BB
    return pl.pallas_call(
        _writer_body,
        grid=(grid,),
        in_specs=[
            pl.BlockSpec((BB, 1), lambda i: (i, 0)),
            pl.BlockSpec((BB, 2, L), lambda i: (i, 0, 0)),
            pl.BlockSpec((L, L), lambda i: (0, 0)),
        ],
        out_specs=pl.BlockSpec((BB, K, L), lambda i: (i, 0, 0)),
        out_shape=jax.ShapeDtypeStruct((B, K, L), jnp.float32),
    )(base2d, logitst, jnp.asarray(_TRI))


def kernel(seq, enref_seq_len, is_new_logits):
    lens = enref_seq_len.astype(jnp.int32)
    seqt = jnp.swapaxes(seq, 1, 2)
    base = _sc_base(seqt, lens)
    logitst = jnp.swapaxes(is_new_logits, 1, 2)
    out_t = _writer(base.reshape(B, 1), logitst)
    out = jnp.swapaxes(out_t, 1, 2)
    return jax.lax.stop_gradient(out)
